# baseline (speedup 1.0000x reference)
import functools

import jax
import jax.numpy as jnp
from jax import lax
from jax.experimental import pallas as pl
from jax.experimental.pallas import tpu as pltpu
from jax.experimental.pallas import tpu_sc as plsc

D = 1000
DP = 1024
B_TOTAL = 81920
NB = 4096
L = 20
NC = 2
NS = 16
NW = NC * NS
B_PER_W = B_TOTAL // NW
CHUNK = 40
N_CHUNKS = B_PER_W // CHUNK


def _mm_body(t_ref, w_ref, b_ref, y_ref):
    acc = lax.dot_general(
        t_ref[...], w_ref[...],
        dimension_numbers=(((1,), (1,)), ((), ())),
        preferred_element_type=jnp.float32,
    ) + b_ref[...]
    y_ref[...] = acc.astype(jnp.bfloat16)


VP = 1008


def _fused_table(table, W, b):
    t_p = jnp.pad(table, ((0, VP - D), (0, 0)))
    w_p = jnp.pad(W, ((0, DP - D), (0, 0)))
    b_p = jnp.pad(b, (0, DP - D))
    return pl.pallas_call(
        _mm_body,
        out_shape=jax.ShapeDtypeStruct((VP, DP), jnp.bfloat16),
    )(t_p, w_p, b_p.reshape(1, DP))


_sc_mesh = plsc.VectorSubcoreMesh(
    core_axis_name="c", subcore_axis_name="s", num_cores=NC, num_subcores=NS
)


@functools.partial(
    pl.kernel,
    out_type=jax.ShapeDtypeStruct((B_TOTAL, DP), jnp.bfloat16),
    mesh=_sc_mesh,
    scratch_types=[
        pltpu.VMEM((B_PER_W,), jnp.int32),
        pltpu.VMEM((CHUNK, DP), jnp.bfloat16),
        pltpu.VMEM((CHUNK, DP), jnp.bfloat16),
        pltpu.SemaphoreType.DMA,
        pltpu.SemaphoreType.DMA,
        pltpu.SemaphoreType.DMA,
        pltpu.SemaphoreType.DMA,
    ],
    compiler_params=pltpu.CompilerParams(
        use_tc_tiling_on_sc=False, needs_layout_passes=True
    ),
)
def _sc_gather(y_hbm, idx_hbm, out_hbm, idx_v, rows_a, rows_b,
               gsem_a, gsem_b, ssem_a, ssem_b):
    wid = lax.axis_index("s") * NC + lax.axis_index("c")
    base = wid * B_PER_W
    pltpu.sync_copy(idx_hbm.at[pl.ds(base, B_PER_W)], idx_v)

    rows = (rows_a, rows_b)
    gsem = (gsem_a, gsem_b)
    ssem = (ssem_a, ssem_b)

    def g_start(c, s):
        pltpu.make_async_copy(
            y_hbm.at[idx_v.at[pl.ds(c * CHUNK, CHUNK)]], rows[s], gsem[s]
        ).start()

    def g_wait(s):
        pltpu.make_async_copy(
            y_hbm.at[idx_v.at[pl.ds(0, CHUNK)]], rows[s], gsem[s]
        ).wait()

    def s_start(c, s):
        pltpu.make_async_copy(
            rows[s], out_hbm.at[pl.ds(base + c * CHUNK, CHUNK)], ssem[s]
        ).start()

    def s_wait(s):
        pltpu.make_async_copy(
            rows[s], out_hbm.at[pl.ds(base, CHUNK)], ssem[s]
        ).wait()

    g_start(0, 0)

    def pair(p, _):
        for s in range(2):
            c = 2 * p + s
            g_wait(s)
            o = 1 - s

            @pl.when(c >= 1)
            def _():
                s_wait(o)

            @pl.when(c + 1 < N_CHUNKS)
            def _():
                g_start(c + 1, o)

            s_start(c, s)
        return 0

    lax.fori_loop(0, N_CHUNKS // 2, pair, 0)
    s_wait((N_CHUNKS - 1) % 2)


_FMT_ROWS = 160


def _fmt_body(in_ref, out_ref):
    v = in_ref[...].astype(jnp.float32)
    for bb in range(_FMT_ROWS // L):
        out_ref[bb] = v[bb * L:(bb + 1) * L, :D]


def _format_out(stage):
    grid = B_TOTAL // _FMT_ROWS
    return pl.pallas_call(
        _fmt_body,
        grid=(grid,),
        in_specs=[pl.BlockSpec((_FMT_ROWS, DP), lambda g: (g, 0))],
        out_specs=pl.BlockSpec((_FMT_ROWS // L, L, D), lambda g: (g, 0, 0)),
        out_shape=jax.ShapeDtypeStruct((NB, L, D), jnp.float32),
    )(stage)


def kernel(x, table, W, b):
    y = _fused_table(table, W, b)
    idx = x.reshape(-1).astype(jnp.int32)
    stage = _sc_gather(y, idx)
    return _format_out(stage)


# speedup vs baseline: 2.0043x; 2.0043x over previous
import functools

import jax
import jax.numpy as jnp
from jax import lax
from jax.experimental import pallas as pl
from jax.experimental.pallas import tpu as pltpu
from jax.experimental.pallas import tpu_sc as plsc

D = 1000
DP = 1024
NB = 4096
L = 20
B_TOTAL = NB * L
NC = 2
NS = 16
NW = NC * NS
B_PER_W = B_TOTAL // NW
CHUNK = 40
N_CHUNKS = B_PER_W // CHUNK


def _mm_body(t_ref, w_ref, b_ref, y_ref):
    y_ref[...] = lax.dot_general(
        t_ref[...], w_ref[...],
        dimension_numbers=(((1,), (1,)), ((), ())),
        preferred_element_type=jnp.float32,
    ) + b_ref[...]


def _fused_table(table, W, b):
    w_p = jnp.pad(W, ((0, DP - D), (0, 0)))
    b_p = jnp.pad(b, (0, DP - D))
    return pl.pallas_call(
        _mm_body,
        out_shape=jax.ShapeDtypeStruct((D, DP), jnp.float32),
    )(table, w_p, b_p.reshape(1, DP))


_sc_mesh = plsc.VectorSubcoreMesh(
    core_axis_name="c", subcore_axis_name="s", num_cores=NC, num_subcores=NS
)


@functools.partial(
    pl.kernel,
    out_type=jax.ShapeDtypeStruct((B_TOTAL, DP), jnp.float32),
    mesh=_sc_mesh,
    scratch_types=[
        pltpu.VMEM((B_PER_W,), jnp.int32),
        pltpu.VMEM((N_CHUNKS, CHUNK), jnp.int32),
        pltpu.VMEM((CHUNK, DP), jnp.float32),
        pltpu.VMEM((CHUNK, DP), jnp.float32),
        pltpu.SemaphoreType.DMA,
        pltpu.SemaphoreType.DMA,
        pltpu.SemaphoreType.DMA,
        pltpu.SemaphoreType.DMA,
    ],
    compiler_params=pltpu.CompilerParams(use_tc_tiling_on_sc=True),
)
def _sc_gather(y_hbm, idx_hbm, dst_hbm, out_hbm, idx_v, dst_v, rows_a, rows_b,
               gsem_a, gsem_b, ssem_a, ssem_b):
    wid = lax.axis_index("s") * NC + lax.axis_index("c")
    base = wid * B_PER_W
    pltpu.sync_copy(idx_hbm.at[pl.ds(base, B_PER_W)], idx_v)
    pltpu.sync_copy(dst_hbm.at[wid], dst_v)

    rows = (rows_a, rows_b)
    gsem = (gsem_a, gsem_b)
    ssem = (ssem_a, ssem_b)

    def g_start(c, s):
        pltpu.make_async_copy(
            y_hbm.at[idx_v.at[pl.ds(c * CHUNK, CHUNK)]], rows[s], gsem[s]
        ).start()

    def g_wait(s):
        pltpu.make_async_copy(
            y_hbm.at[idx_v.at[pl.ds(0, CHUNK)]], rows[s], gsem[s]
        ).wait()

    def s_start(c, s):
        pltpu.make_async_copy(
            rows[s], out_hbm.at[dst_v.at[c]], ssem[s]
        ).start()

    def s_wait(s):
        pltpu.make_async_copy(
            rows[s], out_hbm.at[dst_v.at[0]], ssem[s]
        ).wait()

    g_start(0, 0)

    def pair(p, _):
        for s in range(2):
            c = 2 * p + s
            g_wait(s)
            o = 1 - s

            @pl.when(c >= 1)
            def _():
                s_wait(o)

            @pl.when(c + 1 < N_CHUNKS)
            def _():
                g_start(c + 1, o)

            s_start(c, s)
        return 0

    lax.fori_loop(0, N_CHUNKS // 2, pair, 0)
    s_wait((N_CHUNKS - 1) % 2)


_TB = 128


def _tr_body(in_ref, out_ref):
    out_ref[0] = jnp.transpose(in_ref[...])[:D, :]


def _transpose_out(stage):
    return pl.pallas_call(
        _tr_body,
        grid=(L, NB // _TB),
        in_specs=[pl.BlockSpec((_TB, DP), lambda l, g: (l * (NB // _TB) + g, 0))],
        out_specs=pl.BlockSpec((1, D, _TB), lambda l, g: (l, 0, g)),
        out_shape=jax.ShapeDtypeStruct((L, D, NB), jnp.float32),
    )(stage)


def kernel(x, table, W, b):
    y = _fused_table(table, W, b)
    idx = x.reshape(-1).astype(jnp.int32)
    flat = jnp.arange(B_TOTAL, dtype=jnp.int32)
    dst = ((flat % L) * NB + flat // L).reshape(NW, N_CHUNKS, CHUNK)
    stage = _sc_gather(y, idx, dst)
    out_t = _transpose_out(stage)
    return jnp.transpose(out_t, (2, 0, 1))


# speedup vs baseline: 2.0581x; 1.0268x over previous
import functools

import jax
import jax.numpy as jnp
from jax import lax
from jax.experimental import pallas as pl
from jax.experimental.pallas import tpu as pltpu
from jax.experimental.pallas import tpu_sc as plsc

D = 1000
DP = 1024
NB = 4096
L = 20
B_TOTAL = NB * L
NC = 2
NS = 16
NW = NC * NS
B_PER_W = B_TOTAL // NW
CHUNK = 40
N_CHUNKS = B_PER_W // CHUNK


def _mm_body(t_ref, w_ref, b_ref, y_ref):
    y_ref[...] = lax.dot_general(
        t_ref[...], w_ref[...],
        dimension_numbers=(((1,), (1,)), ((), ())),
        preferred_element_type=jnp.float32,
    ) + b_ref[...]


def _fused_table(table, W, b):
    w_p = jnp.pad(W, ((0, DP - D), (0, 0)))
    b_p = jnp.pad(b, (0, DP - D))
    return pl.pallas_call(
        _mm_body,
        out_shape=jax.ShapeDtypeStruct((D, DP), jnp.float32),
    )(table, w_p, b_p.reshape(1, DP))


_sc_mesh = plsc.VectorSubcoreMesh(
    core_axis_name="c", subcore_axis_name="s", num_cores=NC, num_subcores=NS
)


@functools.partial(
    pl.kernel,
    out_type=jax.ShapeDtypeStruct((B_TOTAL, DP), jnp.float32),
    mesh=_sc_mesh,
    scratch_types=[
        pltpu.VMEM((B_PER_W,), jnp.int32),
        pltpu.VMEM((N_CHUNKS, CHUNK), jnp.int32),
        pltpu.VMEM((CHUNK, DP), jnp.float32),
        pltpu.VMEM((CHUNK, DP), jnp.float32),
        pltpu.SemaphoreType.DMA,
        pltpu.SemaphoreType.DMA,
        pltpu.SemaphoreType.DMA,
        pltpu.SemaphoreType.DMA,
    ],
    compiler_params=pltpu.CompilerParams(use_tc_tiling_on_sc=True),
)
def _sc_gather(y_hbm, idx_hbm, dst_hbm, out_hbm, idx_v, dst_v, rows_a, rows_b,
               gsem_a, gsem_b, ssem_a, ssem_b):
    wid = lax.axis_index("s") * NC + lax.axis_index("c")
    base = wid * B_PER_W
    pltpu.sync_copy(idx_hbm.at[pl.ds(base, B_PER_W)], idx_v)
    pltpu.sync_copy(dst_hbm.at[wid], dst_v)

    rows = (rows_a, rows_b)
    gsem = (gsem_a, gsem_b)
    ssem = (ssem_a, ssem_b)

    def g_start(c, s):
        pltpu.make_async_copy(
            y_hbm.at[idx_v.at[pl.ds(c * CHUNK, CHUNK)]], rows[s], gsem[s]
        ).start()

    def g_wait(s):
        pltpu.make_async_copy(
            y_hbm.at[idx_v.at[pl.ds(0, CHUNK)]], rows[s], gsem[s]
        ).wait()

    def s_start(c, s):
        pltpu.make_async_copy(
            rows[s], out_hbm.at[dst_v.at[c]], ssem[s]
        ).start()

    def s_wait(s):
        pltpu.make_async_copy(
            rows[s], out_hbm.at[dst_v.at[0]], ssem[s]
        ).wait()

    g_start(0, 0)

    def pair(p, _):
        for s in range(2):
            c = 2 * p + s
            g_wait(s)
            o = 1 - s

            @pl.when(c >= 1)
            def _():
                s_wait(o)

            @pl.when(c + 1 < N_CHUNKS)
            def _():
                g_start(c + 1, o)

            s_start(c, s)
        return 0

    lax.fori_loop(0, N_CHUNKS // 2, pair, 0)
    s_wait((N_CHUNKS - 1) % 2)


_TB = 128


def _tr_body(in_ref, out_ref):
    v = in_ref[...].astype(jnp.bfloat16)
    out_ref[0] = jnp.transpose(v)[:D, :].astype(jnp.float32)


def _transpose_out(stage):
    return pl.pallas_call(
        _tr_body,
        grid=(L, NB // _TB),
        in_specs=[pl.BlockSpec((_TB, DP), lambda l, g: (l * (NB // _TB) + g, 0))],
        out_specs=pl.BlockSpec((1, D, _TB), lambda l, g: (l, 0, g)),
        out_shape=jax.ShapeDtypeStruct((L, D, NB), jnp.float32),
    )(stage)


def kernel(x, table, W, b):
    y = _fused_table(table, W, b)
    idx = x.reshape(-1).astype(jnp.int32)
    flat = jnp.arange(B_TOTAL, dtype=jnp.int32)
    dst = ((flat % L) * NB + flat // L).reshape(NW, N_CHUNKS, CHUNK)
    stage = _sc_gather(y, idx, dst)
    out_t = _transpose_out(stage)
    return jnp.transpose(out_t, (2, 0, 1))


# speedup vs baseline: 3.0440x; 1.4790x over previous
import functools

import jax
import jax.numpy as jnp
from jax import lax
from jax.experimental import pallas as pl
from jax.experimental.pallas import tpu as pltpu
from jax.experimental.pallas import tpu_sc as plsc

D = 1000
DP = 1024
NB = 4096
L = 20
B_TOTAL = NB * L
NC = 2
NS = 16
NW = NC * NS
B_PER_W = B_TOTAL // NW
CHUNK = 40
N_CHUNKS = B_PER_W // CHUNK


def _mm_body(t_ref, w_ref, b_ref, y_ref):
    y_ref[...] = lax.dot_general(
        t_ref[...], w_ref[...],
        dimension_numbers=(((1,), (1,)), ((), ())),
        preferred_element_type=jnp.float32,
    ) + b_ref[...]


def _fused_table(table, W, b):
    w_p = jnp.pad(W, ((0, DP - D), (0, 0)))
    b_p = jnp.pad(b, (0, DP - D))
    return pl.pallas_call(
        _mm_body,
        out_shape=jax.ShapeDtypeStruct((D, DP), jnp.float32),
    )(table, w_p, b_p.reshape(1, DP))


_sc_mesh = plsc.VectorSubcoreMesh(
    core_axis_name="c", subcore_axis_name="s", num_cores=NC, num_subcores=NS
)


@functools.partial(
    pl.kernel,
    out_type=jax.ShapeDtypeStruct((B_TOTAL, DP), jnp.float32),
    mesh=_sc_mesh,
    scratch_types=[
        pltpu.VMEM((B_PER_W,), jnp.int32),
        pltpu.VMEM((N_CHUNKS, CHUNK), jnp.int32),
        pltpu.VMEM((CHUNK, DP), jnp.float32),
        pltpu.VMEM((CHUNK, DP), jnp.float32),
        pltpu.SemaphoreType.DMA,
        pltpu.SemaphoreType.DMA,
        pltpu.SemaphoreType.DMA,
        pltpu.SemaphoreType.DMA,
    ],
    compiler_params=pltpu.CompilerParams(use_tc_tiling_on_sc=True),
)
def _sc_gather(y_hbm, idx_hbm, dst_hbm, out_hbm, idx_v, dst_v, rows_a, rows_b,
               gsem_a, gsem_b, ssem_a, ssem_b):
    wid = lax.axis_index("s") * NC + lax.axis_index("c")
    base = wid * B_PER_W
    pltpu.sync_copy(idx_hbm.at[pl.ds(base, B_PER_W)], idx_v)
    pltpu.sync_copy(dst_hbm.at[wid], dst_v)

    rows = (rows_a, rows_b)
    gsem = (gsem_a, gsem_b)
    ssem = (ssem_a, ssem_b)

    def g_start(c, s):
        pltpu.make_async_copy(
            y_hbm.at[idx_v.at[pl.ds(c * CHUNK, CHUNK)]], rows[s], gsem[s]
        ).start()

    def g_wait(s):
        pltpu.make_async_copy(
            y_hbm.at[idx_v.at[pl.ds(0, CHUNK)]], rows[s], gsem[s]
        ).wait()

    def s_start(c, s):
        pltpu.make_async_copy(
            rows[s], out_hbm.at[dst_v.at[c]], ssem[s]
        ).start()

    def s_wait(s):
        pltpu.make_async_copy(
            rows[s], out_hbm.at[dst_v.at[0]], ssem[s]
        ).wait()

    g_start(0, 0)

    def pair(p, _):
        for s in range(2):
            c = 2 * p + s
            g_wait(s)
            o = 1 - s

            @pl.when(c >= 1)
            def _():
                s_wait(o)

            @pl.when(c + 1 < N_CHUNKS)
            def _():
                g_start(c + 1, o)

            s_start(c, s)
        return 0

    lax.fori_loop(0, N_CHUNKS // 2, pair, 0)
    s_wait((N_CHUNKS - 1) % 2)


_TB = 512


def _tr_body(in_ref, out_ref):
    v = in_ref[...].astype(jnp.bfloat16)
    out_ref[0] = jnp.transpose(v)[:D, :].astype(jnp.float32)


def _transpose_out(stage):
    return pl.pallas_call(
        _tr_body,
        grid=(L, NB // _TB),
        in_specs=[pl.BlockSpec((_TB, DP), lambda l, g: (l * (NB // _TB) + g, 0))],
        out_specs=pl.BlockSpec((1, D, _TB), lambda l, g: (l, 0, g)),
        out_shape=jax.ShapeDtypeStruct((L, D, NB), jnp.float32),
    )(stage)


def kernel(x, table, W, b):
    y = _fused_table(table, W, b)
    idx = x.reshape(-1).astype(jnp.int32)
    flat = jnp.arange(B_TOTAL, dtype=jnp.int32)
    dst = ((flat % L) * NB + flat // L).reshape(NW, N_CHUNKS, CHUNK)
    stage = _sc_gather(y, idx, dst)
    out_t = _transpose_out(stage)
    return jnp.transpose(out_t, (2, 0, 1))


# speedup vs baseline: 3.2205x; 1.0580x over previous
import functools

import jax
import jax.numpy as jnp
from jax import lax
from jax.experimental import pallas as pl
from jax.experimental.pallas import tpu as pltpu
from jax.experimental.pallas import tpu_sc as plsc

D = 1000
DP = 1024
NB = 4096
L = 20
B_TOTAL = NB * L
NC = 2
NS = 16
NW = NC * NS
B_PER_W = B_TOTAL // NW
CHUNK = 40
N_CHUNKS = B_PER_W // CHUNK


def _mm_body(t_ref, w_ref, b_ref, y_ref):
    y_ref[...] = lax.dot_general(
        t_ref[...], w_ref[...],
        dimension_numbers=(((1,), (1,)), ((), ())),
        preferred_element_type=jnp.float32,
    ) + b_ref[...]


def _fused_table(table, W, b):
    w_p = jnp.pad(W, ((0, DP - D), (0, 0)))
    b_p = jnp.pad(b, (0, DP - D))
    return pl.pallas_call(
        _mm_body,
        out_shape=jax.ShapeDtypeStruct((D, DP), jnp.float32),
    )(table, w_p, b_p.reshape(1, DP))


_sc_mesh = plsc.VectorSubcoreMesh(
    core_axis_name="c", subcore_axis_name="s", num_cores=NC, num_subcores=NS
)


@functools.partial(
    pl.kernel,
    out_type=jax.ShapeDtypeStruct((B_TOTAL, DP), jnp.float32),
    mesh=_sc_mesh,
    scratch_types=[
        pltpu.VMEM((B_PER_W,), jnp.int32),
        pltpu.VMEM((N_CHUNKS, CHUNK), jnp.int32),
        pltpu.VMEM((CHUNK, DP), jnp.float32),
        pltpu.VMEM((CHUNK, DP), jnp.float32),
        pltpu.SemaphoreType.DMA,
        pltpu.SemaphoreType.DMA,
        pltpu.SemaphoreType.DMA,
        pltpu.SemaphoreType.DMA,
    ],
    compiler_params=pltpu.CompilerParams(use_tc_tiling_on_sc=True),
)
def _sc_gather(y_hbm, idx_hbm, dst_hbm, out_hbm, idx_v, dst_v, rows_a, rows_b,
               gsem_a, gsem_b, ssem_a, ssem_b):
    wid = lax.axis_index("s") * NC + lax.axis_index("c")
    base = wid * B_PER_W
    pltpu.sync_copy(idx_hbm.at[pl.ds(base, B_PER_W)], idx_v)
    pltpu.sync_copy(dst_hbm.at[wid], dst_v)

    rows = (rows_a, rows_b)
    gsem = (gsem_a, gsem_b)
    ssem = (ssem_a, ssem_b)

    def g_start(c, s):
        pltpu.make_async_copy(
            y_hbm.at[idx_v.at[pl.ds(c * CHUNK, CHUNK)]], rows[s], gsem[s]
        ).start()

    def g_wait(s):
        pltpu.make_async_copy(
            y_hbm.at[idx_v.at[pl.ds(0, CHUNK)]], rows[s], gsem[s]
        ).wait()

    def s_start(c, s):
        pltpu.make_async_copy(
            rows[s], out_hbm.at[dst_v.at[c]], ssem[s]
        ).start()

    def s_wait(s):
        pltpu.make_async_copy(
            rows[s], out_hbm.at[dst_v.at[0]], ssem[s]
        ).wait()

    g_start(0, 0)

    def pair(p, _):
        for s in range(2):
            c = 2 * p + s
            g_wait(s)
            o = 1 - s

            @pl.when(c >= 1)
            def _():
                s_wait(o)

            @pl.when(c + 1 < N_CHUNKS)
            def _():
                g_start(c + 1, o)

            s_start(c, s)
        return 0

    lax.fori_loop(0, N_CHUNKS // 2, pair, 0)
    s_wait((N_CHUNKS - 1) % 2)


_TB = 1024


def _tr_body(in_ref, out_ref):
    v = in_ref[...].astype(jnp.bfloat16)
    out_ref[0] = jnp.transpose(v)[:D, :].astype(jnp.float32)


def _transpose_out(stage):
    return pl.pallas_call(
        _tr_body,
        grid=(L, NB // _TB),
        in_specs=[pl.BlockSpec((_TB, DP), lambda l, g: (l * (NB // _TB) + g, 0))],
        out_specs=pl.BlockSpec((1, D, _TB), lambda l, g: (l, 0, g)),
        out_shape=jax.ShapeDtypeStruct((L, D, NB), jnp.float32),
    )(stage)


def kernel(x, table, W, b):
    y = _fused_table(table, W, b)
    idx = x.reshape(-1).astype(jnp.int32)
    flat = jnp.arange(B_TOTAL, dtype=jnp.int32)
    dst = ((flat % L) * NB + flat // L).reshape(NW, N_CHUNKS, CHUNK)
    stage = _sc_gather(y, idx, dst)
    out_t = _transpose_out(stage)
    return jnp.transpose(out_t, (2, 0, 1))
